# Initial kernel scaffold; baseline (speedup 1.0000x reference)
import functools

import jax
import jax.numpy as jnp
from jax import lax
from jax.experimental import pallas as pl
from jax.experimental.pallas import tpu as pltpu
from jax.experimental.pallas import tpu_sc as plsc

N_NODES = 10000
D = 128
N_EDGES = 320000
N_TILES = 32
CHUNK = 128
CHUNKS_PER_TILE = 80
E_PAD = N_TILES * CHUNKS_PER_TILE * CHUNK
ACC_ROWS = 10112
ROWS_PER_TILE = ACC_ROWS // 16

_mesh = plsc.VectorSubcoreMesh(core_axis_name="c", subcore_axis_name="s")


@functools.partial(
    pl.kernel,
    out_type=jax.ShapeDtypeStruct((2, ACC_ROWS), jnp.float32),
    mesh=_mesh,
    scratch_types=[
        pltpu.VMEM((CHUNKS_PER_TILE, CHUNK), jnp.int32),
        pltpu.VMEM((CHUNK,), jnp.float32),
        pltpu.VMEM_SHARED((ACC_ROWS,), jnp.float32),
        pltpu.SemaphoreType.DMA,
    ],
)
def _deg_kernel(dst_hbm, zdeg_hbm, out_hbm, dstv, onesv, degsh, dsem):
    cid = lax.axis_index("c")
    sid = lax.axis_index("s")
    wid = cid * 16 + sid
    pltpu.sync_copy(dst_hbm.at[wid], dstv)
    for k in range(8):
        onesv[pl.ds(k * 16, 16)] = jnp.ones((16,), jnp.float32)
    pltpu.sync_copy(
        zdeg_hbm.at[pl.ds(sid * ROWS_PER_TILE, ROWS_PER_TILE)],
        degsh.at[pl.ds(sid * ROWS_PER_TILE, ROWS_PER_TILE)],
    )
    plsc.subcore_barrier()

    def body(t, carry):
        for k in range(8):
            pltpu.async_copy(onesv, degsh.at[dstv.at[t * 8 + k]], dsem, add=True)
        for k in range(8):
            pltpu.make_async_copy(onesv, degsh.at[dstv.at[0]], dsem).wait()
        return carry

    lax.fori_loop(0, CHUNKS_PER_TILE // 8, body, 0)
    plsc.subcore_barrier()
    pltpu.sync_copy(
        degsh.at[pl.ds(sid * ROWS_PER_TILE, ROWS_PER_TILE)],
        out_hbm.at[cid, pl.ds(sid * ROWS_PER_TILE, ROWS_PER_TILE)],
    )


@functools.partial(
    pl.kernel,
    out_type=jax.ShapeDtypeStruct((2, ACC_ROWS, D), jnp.float32),
    mesh=_mesh,
    scratch_types=[
        pltpu.VMEM((CHUNKS_PER_TILE, CHUNK), jnp.int32),
        pltpu.VMEM((CHUNKS_PER_TILE, CHUNK), jnp.int32),
        pltpu.VMEM((CHUNK, D), jnp.float32),
        pltpu.VMEM((CHUNK, D), jnp.float32),
        pltpu.VMEM((CHUNK, D), jnp.float32),
        pltpu.VMEM((CHUNK, D), jnp.float32),
        pltpu.VMEM_SHARED((ACC_ROWS, D), jnp.float32),
        pltpu.SemaphoreType.DMA,
        pltpu.SemaphoreType.DMA,
        pltpu.SemaphoreType.DMA,
        pltpu.SemaphoreType.DMA,
        pltpu.SemaphoreType.DMA,
        pltpu.SemaphoreType.DMA,
        pltpu.SemaphoreType.DMA,
        pltpu.SemaphoreType.DMA,
    ],
)
def _agg_kernel(h_hbm, src_hbm, dst_hbm, zrows_hbm, out_hbm,
                srcv, dstv, b0, b1, b2, b3, acc,
                g0, g1, g2, g3, s0, s1, s2, s3):
    cid = lax.axis_index("c")
    sid = lax.axis_index("s")
    wid = cid * 16 + sid
    bufs = (b0, b1, b2, b3)
    gsems = (g0, g1, g2, g3)
    ssems = (s0, s1, s2, s3)

    pltpu.sync_copy(src_hbm.at[wid], srcv)
    pltpu.sync_copy(dst_hbm.at[wid], dstv)
    pltpu.sync_copy(
        zrows_hbm.at[pl.ds(sid * ROWS_PER_TILE, ROWS_PER_TILE)],
        acc.at[pl.ds(sid * ROWS_PER_TILE, ROWS_PER_TILE)],
    )
    plsc.subcore_barrier()

    for b in range(4):
        pltpu.async_copy(h_hbm.at[srcv.at[b]], bufs[b], gsems[b])

    n_outer = CHUNKS_PER_TILE // 4

    def body(t, carry):
        for b in range(4):
            c = t * 4 + b
            pltpu.make_async_copy(h_hbm.at[srcv.at[0]], bufs[b], gsems[b]).wait()
            pltpu.async_copy(bufs[b], acc.at[dstv.at[c]], ssems[b], add=True)

            @pl.when(t < n_outer - 1)
            def _():
                pltpu.make_async_copy(bufs[b], acc.at[dstv.at[0]], ssems[b]).wait()
                pltpu.async_copy(h_hbm.at[srcv.at[c + 4]], bufs[b], gsems[b])

        return carry

    lax.fori_loop(0, n_outer, body, 0)
    for b in range(4):
        pltpu.make_async_copy(bufs[b], acc.at[dstv.at[0]], ssems[b]).wait()
    plsc.subcore_barrier()
    pltpu.sync_copy(
        acc.at[pl.ds(sid * ROWS_PER_TILE, ROWS_PER_TILE)],
        out_hbm.at[cid, pl.ds(sid * ROWS_PER_TILE, ROWS_PER_TILE)],
    )


def _dinv_body(degp_ref, o_ref):
    o_ref[...] = lax.rsqrt(degp_ref[0] + degp_ref[1] + 1.0)


def _mm_scale_body(x_ref, w_ref, dv_ref, o_ref):
    o_ref[...] = jnp.dot(x_ref[...], w_ref[...],
                         preferred_element_type=jnp.float32) * dv_ref[...]


def _mid_body(ap_ref, hp_ref, dv_ref, b_ref, w_ref, o_ref):
    t = dv_ref[...] * (ap_ref[0] + ap_ref[1] + hp_ref[...]) + b_ref[...]
    h = jnp.maximum(t, 0.0)
    o_ref[...] = jnp.dot(h, w_ref[...],
                         preferred_element_type=jnp.float32) * dv_ref[...]


def _fin_body(ap_ref, hp_ref, dv_ref, b_ref, o_ref):
    o_ref[...] = dv_ref[...] * (ap_ref[0] + ap_ref[1] + hp_ref[...]) + b_ref[...]


_RB = 1000


def kernel(x, edge_index, W1, b1, W2, b2):
    src = edge_index[0].astype(jnp.int32)
    dst = edge_index[1].astype(jnp.int32)
    pad = jnp.full((E_PAD - N_EDGES,), N_NODES, jnp.int32)
    srcg = jnp.concatenate([src, pad]).reshape(N_TILES, CHUNKS_PER_TILE, CHUNK)
    dstg = jnp.concatenate([dst, pad]).reshape(N_TILES, CHUNKS_PER_TILE, CHUNK)
    zdeg = jnp.zeros((ACC_ROWS,), jnp.float32)
    zrows = jnp.zeros((ACC_ROWS, D), jnp.float32)
    pad_rows = jnp.zeros((ACC_ROWS - N_NODES, D), jnp.float32)

    degp = _deg_kernel(dstg, zdeg)
    dinv2d = pl.pallas_call(
        _dinv_body,
        out_shape=jax.ShapeDtypeStruct((ACC_ROWS // D, D), jnp.float32),
    )(degp.reshape(2, ACC_ROWS // D, D))
    dinv_col = dinv2d.reshape(ACC_ROWS)[:N_NODES][:, None]

    h1p = pl.pallas_call(
        _mm_scale_body,
        grid=(N_NODES // _RB,),
        in_specs=[
            pl.BlockSpec((_RB, D), lambda i: (i, 0)),
            pl.BlockSpec((D, D), lambda i: (0, 0)),
            pl.BlockSpec((_RB, 1), lambda i: (i, 0)),
        ],
        out_specs=pl.BlockSpec((_RB, D), lambda i: (i, 0)),
        out_shape=jax.ShapeDtypeStruct((N_NODES, D), jnp.float32),
    )(x, W1, dinv_col)
    h1ext = jnp.concatenate([h1p, pad_rows])
    acc1 = _agg_kernel(h1ext, srcg, dstg, zrows)

    h2p = pl.pallas_call(
        _mid_body,
        grid=(N_NODES // _RB,),
        in_specs=[
            pl.BlockSpec((2, _RB, D), lambda i: (0, i, 0)),
            pl.BlockSpec((_RB, D), lambda i: (i, 0)),
            pl.BlockSpec((_RB, 1), lambda i: (i, 0)),
            pl.BlockSpec((1, D), lambda i: (0, 0)),
            pl.BlockSpec((D, D), lambda i: (0, 0)),
        ],
        out_specs=pl.BlockSpec((_RB, D), lambda i: (i, 0)),
        out_shape=jax.ShapeDtypeStruct((N_NODES, D), jnp.float32),
    )(acc1, h1p, dinv_col, b1.reshape(1, D), W2)
    h2ext = jnp.concatenate([h2p, pad_rows])
    acc2 = _agg_kernel(h2ext, srcg, dstg, zrows)

    out = pl.pallas_call(
        _fin_body,
        grid=(N_NODES // _RB,),
        in_specs=[
            pl.BlockSpec((2, _RB, D), lambda i: (0, i, 0)),
            pl.BlockSpec((_RB, D), lambda i: (i, 0)),
            pl.BlockSpec((_RB, 1), lambda i: (i, 0)),
            pl.BlockSpec((1, D), lambda i: (0, 0)),
        ],
        out_specs=pl.BlockSpec((_RB, D), lambda i: (i, 0)),
        out_shape=jax.ShapeDtypeStruct((N_NODES, D), jnp.float32),
    )(acc2, h2p, dinv_col, b2.reshape(1, D))
    return out


# speedup vs baseline: 5.7017x; 5.7017x over previous
import functools

import jax
import jax.numpy as jnp
from jax import lax
from jax.experimental import pallas as pl
from jax.experimental.pallas import tpu as pltpu
from jax.experimental.pallas import tpu_sc as plsc

N_NODES = 10000
D = 128
N_EDGES = 320000
N_TILES = 32
CHUNK = 128
DEG_CHUNKS = 80
AGG_CHUNKS = 160
E_PAD = N_TILES * DEG_CHUNKS * CHUNK
ACC_ROWS = 10112
ROWS_PER_TILE = ACC_ROWS // 16
OWN = 5056
ACC_HALF = 5120
HROWS_PER_TILE = ACC_HALF // 16

_mesh = plsc.VectorSubcoreMesh(core_axis_name="c", subcore_axis_name="s")


@functools.partial(
    pl.kernel,
    out_type=jax.ShapeDtypeStruct((2, ACC_ROWS, D), jnp.float32),
    mesh=_mesh,
    scratch_types=[
        pltpu.VMEM((DEG_CHUNKS, CHUNK), jnp.int32),
        pltpu.VMEM((16, D), jnp.float32),
        pltpu.VMEM((CHUNK, D), jnp.float32),
        pltpu.VMEM((16, D), jnp.float32),
        pltpu.VMEM_SHARED((16, D), jnp.float32),
        pltpu.VMEM_SHARED((ACC_ROWS, D), jnp.float32),
        pltpu.SemaphoreType.DMA,
        pltpu.SemaphoreType.DMA,
    ],
)
def _deg_kernel(dst_hbm, out_hbm, dstv, ones16, onesb, zb, ones_sh, deg,
                zsem, dsem):
    cid = lax.axis_index("c")
    sid = lax.axis_index("s")
    wid = cid * 16 + sid
    pltpu.sync_copy(dst_hbm.at[wid], dstv)
    for i in range(16):
        for k in range(8):
            zb[i, pl.ds(k * 16, 16)] = jnp.zeros((16,), jnp.float32)

    @pl.when(sid == 0)
    def _():
        for i in range(16):
            for k in range(8):
                ones16[i, pl.ds(k * 16, 16)] = jnp.ones((16,), jnp.float32)
        pltpu.sync_copy(ones16, ones_sh)

    base = sid * ROWS_PER_TILE
    nz = ROWS_PER_TILE // 16
    for j in range(nz):
        pltpu.async_copy(zb, deg.at[pl.ds(base + j * 16, 16)], zsem)
    pltpu.async_copy(zb, deg.at[pl.ds(base + ROWS_PER_TILE - 16, 16)], zsem)
    for j in range(nz + 1):
        pltpu.make_async_copy(zb, deg.at[pl.ds(0, 16)], zsem).wait()
    plsc.subcore_barrier()
    for j in range(CHUNK // 16):
        pltpu.async_copy(ones_sh, onesb.at[pl.ds(j * 16, 16)], zsem)
    for j in range(CHUNK // 16):
        pltpu.make_async_copy(ones_sh, onesb.at[pl.ds(0, 16)], zsem).wait()

    def body(t, carry):
        for k in range(4):
            pltpu.async_copy(onesb, deg.at[dstv.at[t * 4 + k]], dsem, add=True)
        for k in range(4):
            pltpu.make_async_copy(onesb, deg.at[dstv.at[0]], dsem).wait()
        return carry

    lax.fori_loop(0, DEG_CHUNKS // 4, body, 0)
    plsc.subcore_barrier()
    pltpu.sync_copy(
        deg.at[pl.ds(sid * ROWS_PER_TILE, ROWS_PER_TILE)],
        out_hbm.at[cid, pl.ds(sid * ROWS_PER_TILE, ROWS_PER_TILE)],
    )


@functools.partial(
    pl.kernel,
    out_type=jax.ShapeDtypeStruct((2, ACC_HALF, D), jnp.float32),
    mesh=_mesh,
    scratch_types=[
        pltpu.VMEM((AGG_CHUNKS, CHUNK), jnp.int32),
        pltpu.VMEM((AGG_CHUNKS, CHUNK), jnp.int32),
        pltpu.VMEM((CHUNK, D), jnp.float32),
        pltpu.VMEM((CHUNK, D), jnp.float32),
        pltpu.VMEM((16, D), jnp.float32),
        pltpu.VMEM_SHARED((ACC_HALF, D), jnp.float32),
        pltpu.SemaphoreType.DMA,
        pltpu.SemaphoreType.DMA,
        pltpu.SemaphoreType.DMA,
        pltpu.SemaphoreType.DMA,
        pltpu.SemaphoreType.DMA,
    ],
)
def _agg_kernel(h_hbm, src_hbm, dst_hbm, out_hbm,
                srcv, dstv, b0, b1, zb, acc,
                g0, g1, s0, s1, zsem):
    cid = lax.axis_index("c")
    sid = lax.axis_index("s")
    bufs = (b0, b1)
    gsems = (g0, g1)
    ssems = (s0, s1)

    pltpu.sync_copy(src_hbm.at[sid], srcv)
    pltpu.sync_copy(dst_hbm.at[cid, sid], dstv)
    for i in range(16):
        for k in range(8):
            zb[i, pl.ds(k * 16, 16)] = jnp.zeros((16,), jnp.float32)
    base = sid * HROWS_PER_TILE
    for j in range(HROWS_PER_TILE // 16):
        pltpu.async_copy(zb, acc.at[pl.ds(base + j * 16, 16)], zsem)
    for j in range(HROWS_PER_TILE // 16):
        pltpu.make_async_copy(zb, acc.at[pl.ds(0, 16)], zsem).wait()
    plsc.subcore_barrier()

    for b in range(2):
        pltpu.async_copy(h_hbm.at[srcv.at[b]], bufs[b], gsems[b])

    n_outer = AGG_CHUNKS // 2

    def body(t, carry):
        for b in range(2):
            c = t * 2 + b
            pltpu.make_async_copy(h_hbm.at[srcv.at[0]], bufs[b], gsems[b]).wait()
            pltpu.async_copy(bufs[b], acc.at[dstv.at[c]], ssems[b], add=True)

            @pl.when(t < n_outer - 1)
            def _():
                pltpu.make_async_copy(bufs[b], acc.at[dstv.at[0]], ssems[b]).wait()
                pltpu.async_copy(h_hbm.at[srcv.at[c + 2]], bufs[b], gsems[b])

        return carry

    lax.fori_loop(0, n_outer, body, 0)
    for b in range(2):
        pltpu.make_async_copy(bufs[b], acc.at[dstv.at[0]], ssems[b]).wait()
    plsc.subcore_barrier()
    pltpu.sync_copy(
        acc.at[pl.ds(sid * HROWS_PER_TILE, HROWS_PER_TILE)],
        out_hbm.at[cid, pl.ds(sid * HROWS_PER_TILE, HROWS_PER_TILE)],
    )


def _dinv_body(degp_ref, o_ref):
    o_ref[...] = lax.rsqrt(degp_ref[0] + degp_ref[1] + 1.0)


def _mm_scale_body(x_ref, w_ref, dv_ref, o_ref):
    o_ref[...] = jnp.dot(x_ref[...], w_ref[...],
                         preferred_element_type=jnp.float32) * dv_ref[...]


def _mid_body(ap_ref, hp_ref, dv_ref, b_ref, w_ref, o_ref):
    t = dv_ref[...] * (ap_ref[...] + hp_ref[...]) + b_ref[...]
    h = jnp.maximum(t, 0.0)
    o_ref[...] = jnp.dot(h, w_ref[...],
                         preferred_element_type=jnp.float32) * dv_ref[...]


def _fin_body(ap_ref, hp_ref, dv_ref, b_ref, o_ref):
    o_ref[...] = dv_ref[...] * (ap_ref[...] + hp_ref[...]) + b_ref[...]


_RB = 1000


def kernel(x, edge_index, W1, b1, W2, b2):
    src = edge_index[0].astype(jnp.int32)
    dst = edge_index[1].astype(jnp.int32)
    pad = jnp.full((E_PAD - N_EDGES,), N_NODES, jnp.int32)
    src_pad = jnp.concatenate([src, pad])
    dst_pad = jnp.concatenate([dst, pad])
    dstg32 = dst_pad.reshape(N_TILES, DEG_CHUNKS, CHUNK)
    srcg = src_pad.reshape(16, AGG_CHUNKS, CHUNK)
    dst0 = jnp.where(dst_pad < OWN, dst_pad, OWN)
    dst1 = jnp.where(dst_pad >= OWN, dst_pad - OWN, OWN)
    dstg = jnp.stack([dst0, dst1]).reshape(2, 16, AGG_CHUNKS, CHUNK)
    pad_rows = jnp.zeros((ACC_ROWS - N_NODES, D), jnp.float32)

    degp = _deg_kernel(dstg32)[:, :, 0]
    dinv2d = pl.pallas_call(
        _dinv_body,
        out_shape=jax.ShapeDtypeStruct((ACC_ROWS // D, D), jnp.float32),
    )(degp.reshape(2, ACC_ROWS // D, D))
    dinv_col = dinv2d.reshape(ACC_ROWS)[:N_NODES][:, None]

    h1p = pl.pallas_call(
        _mm_scale_body,
        grid=(N_NODES // _RB,),
        in_specs=[
            pl.BlockSpec((_RB, D), lambda i: (i, 0)),
            pl.BlockSpec((D, D), lambda i: (0, 0)),
            pl.BlockSpec((_RB, 1), lambda i: (i, 0)),
        ],
        out_specs=pl.BlockSpec((_RB, D), lambda i: (i, 0)),
        out_shape=jax.ShapeDtypeStruct((N_NODES, D), jnp.float32),
    )(x, W1, dinv_col)
    h1ext = jnp.concatenate([h1p, pad_rows])
    acc1 = _agg_kernel(h1ext, srcg, dstg)
    agg1 = jnp.concatenate([acc1[0, :OWN], acc1[1, :N_NODES - OWN]])

    h2p = pl.pallas_call(
        _mid_body,
        grid=(N_NODES // _RB,),
        in_specs=[
            pl.BlockSpec((_RB, D), lambda i: (i, 0)),
            pl.BlockSpec((_RB, D), lambda i: (i, 0)),
            pl.BlockSpec((_RB, 1), lambda i: (i, 0)),
            pl.BlockSpec((1, D), lambda i: (0, 0)),
            pl.BlockSpec((D, D), lambda i: (0, 0)),
        ],
        out_specs=pl.BlockSpec((_RB, D), lambda i: (i, 0)),
        out_shape=jax.ShapeDtypeStruct((N_NODES, D), jnp.float32),
    )(agg1, h1p, dinv_col, b1.reshape(1, D), W2)
    h2ext = jnp.concatenate([h2p, pad_rows])
    acc2 = _agg_kernel(h2ext, srcg, dstg)
    agg2 = jnp.concatenate([acc2[0, :OWN], acc2[1, :N_NODES - OWN]])

    out = pl.pallas_call(
        _fin_body,
        grid=(N_NODES // _RB,),
        in_specs=[
            pl.BlockSpec((_RB, D), lambda i: (i, 0)),
            pl.BlockSpec((_RB, D), lambda i: (i, 0)),
            pl.BlockSpec((_RB, 1), lambda i: (i, 0)),
            pl.BlockSpec((1, D), lambda i: (0, 0)),
        ],
        out_specs=pl.BlockSpec((_RB, D), lambda i: (i, 0)),
        out_shape=jax.ShapeDtypeStruct((N_NODES, D), jnp.float32),
    )(agg2, h2p, dinv_col, b2.reshape(1, D))
    return out


# speedup vs baseline: 9.8633x; 1.7299x over previous
import functools

import jax
import jax.numpy as jnp
from jax import lax
from jax.experimental import pallas as pl
from jax.experimental.pallas import tpu as pltpu
from jax.experimental.pallas import tpu_sc as plsc

N_NODES = 10000
D = 128
N_EDGES = 320000
N_TILES = 32
CHUNK = 128
DEG_CHUNKS = 80
AGG_CHUNKS = 80
HALF_CHUNKS = AGG_CHUNKS // 2
E_PAD = N_TILES * DEG_CHUNKS * CHUNK
ACC_ROWS = 10112
ROWS_PER_TILE = ACC_ROWS // 16

_mesh = plsc.VectorSubcoreMesh(core_axis_name="c", subcore_axis_name="s")


@functools.partial(
    pl.kernel,
    out_type=jax.ShapeDtypeStruct((2, ACC_ROWS, D), jnp.float32),
    mesh=_mesh,
    scratch_types=[
        pltpu.VMEM((DEG_CHUNKS, CHUNK), jnp.int32),
        pltpu.VMEM((16, D), jnp.float32),
        pltpu.VMEM((CHUNK, D), jnp.float32),
        pltpu.VMEM((16, D), jnp.float32),
        pltpu.VMEM_SHARED((16, D), jnp.float32),
        pltpu.VMEM_SHARED((ACC_ROWS, D), jnp.float32),
        pltpu.SemaphoreType.DMA,
        pltpu.SemaphoreType.DMA,
    ],
)
def _deg_kernel(dst_hbm, out_hbm, dstv, ones16, onesb, zb, ones_sh, deg,
                zsem, dsem):
    cid = lax.axis_index("c")
    sid = lax.axis_index("s")
    wid = cid * 16 + sid
    pltpu.sync_copy(dst_hbm.at[wid], dstv)
    for i in range(16):
        for k in range(8):
            zb[i, pl.ds(k * 16, 16)] = jnp.zeros((16,), jnp.float32)

    @pl.when(sid == 0)
    def _():
        for i in range(16):
            for k in range(8):
                ones16[i, pl.ds(k * 16, 16)] = jnp.ones((16,), jnp.float32)
        pltpu.sync_copy(ones16, ones_sh)

    base = sid * ROWS_PER_TILE
    nz = ROWS_PER_TILE // 16
    for j in range(nz):
        pltpu.async_copy(zb, deg.at[pl.ds(base + j * 16, 16)], zsem)
    pltpu.async_copy(zb, deg.at[pl.ds(base + ROWS_PER_TILE - 16, 16)], zsem)
    for j in range(nz + 1):
        pltpu.make_async_copy(zb, deg.at[pl.ds(0, 16)], zsem).wait()
    plsc.subcore_barrier()
    for j in range(CHUNK // 16):
        pltpu.async_copy(ones_sh, onesb.at[pl.ds(j * 16, 16)], zsem)
    for j in range(CHUNK // 16):
        pltpu.make_async_copy(ones_sh, onesb.at[pl.ds(0, 16)], zsem).wait()

    def body(t, carry):
        for k in range(4):
            pltpu.async_copy(onesb, deg.at[dstv.at[t * 4 + k]], dsem, add=True)
        for k in range(4):
            pltpu.make_async_copy(onesb, deg.at[dstv.at[0]], dsem).wait()
        return carry

    lax.fori_loop(0, DEG_CHUNKS // 4, body, 0)
    plsc.subcore_barrier()
    pltpu.sync_copy(
        deg.at[pl.ds(sid * ROWS_PER_TILE, ROWS_PER_TILE)],
        out_hbm.at[cid, pl.ds(sid * ROWS_PER_TILE, ROWS_PER_TILE)],
    )


@functools.partial(
    pl.kernel,
    out_type=jax.ShapeDtypeStruct((2, ACC_ROWS, D), jnp.float32),
    mesh=_mesh,
    scratch_types=[
        pltpu.VMEM((HALF_CHUNKS, CHUNK), jnp.int32),
        pltpu.VMEM((HALF_CHUNKS, CHUNK), jnp.int32),
        pltpu.VMEM((CHUNK, D), jnp.float32),
        pltpu.VMEM((CHUNK, D), jnp.float32),
        pltpu.VMEM((16, D), jnp.float32),
        pltpu.VMEM_SHARED((ACC_ROWS, D), jnp.float32),
        pltpu.SemaphoreType.DMA,
        pltpu.SemaphoreType.DMA,
        pltpu.SemaphoreType.DMA,
        pltpu.SemaphoreType.DMA,
        pltpu.SemaphoreType.DMA,
    ],
)
def _agg_kernel(h_hbm, src_hbm, dst_hbm, out_hbm,
                srcv, dstv, b0, b1, zb, acc,
                g0, g1, s0, s1, zsem):
    cid = lax.axis_index("c")
    sid = lax.axis_index("s")
    wid = cid * 16 + sid
    bufs = (b0, b1)
    gsems = (g0, g1)
    ssems = (s0, s1)

    for i in range(16):
        for k in range(8):
            zb[i, pl.ds(k * 16, 16)] = jnp.zeros((16,), jnp.float32)
    base = sid * ROWS_PER_TILE
    nz = ROWS_PER_TILE // 16
    for j in range(nz):
        pltpu.async_copy(zb, acc.at[pl.ds(base + j * 16, 16)], zsem)
    pltpu.async_copy(zb, acc.at[pl.ds(base + ROWS_PER_TILE - 16, 16)], zsem)
    for j in range(nz + 1):
        pltpu.make_async_copy(zb, acc.at[pl.ds(0, 16)], zsem).wait()
    plsc.subcore_barrier()

    n_outer = HALF_CHUNKS // 2

    for p in range(2):
        pltpu.sync_copy(src_hbm.at[wid, p], srcv)
        pltpu.sync_copy(dst_hbm.at[wid, p], dstv)
        for b in range(2):
            pltpu.async_copy(h_hbm.at[srcv.at[b]], bufs[b], gsems[b])

        def body(t, carry):
            for b in range(2):
                c = t * 2 + b
                pltpu.make_async_copy(h_hbm.at[srcv.at[0]], bufs[b],
                                      gsems[b]).wait()
                pltpu.async_copy(bufs[b], acc.at[dstv.at[c]], ssems[b],
                                 add=True)

                @pl.when(t < n_outer - 1)
                def _():
                    pltpu.make_async_copy(bufs[b], acc.at[dstv.at[0]],
                                          ssems[b]).wait()
                    pltpu.async_copy(h_hbm.at[srcv.at[c + 2]], bufs[b],
                                     gsems[b])

            return carry

        lax.fori_loop(0, n_outer, body, 0)
        for b in range(2):
            pltpu.make_async_copy(bufs[b], acc.at[dstv.at[0]], ssems[b]).wait()
    plsc.subcore_barrier()
    pltpu.sync_copy(
        acc.at[pl.ds(sid * ROWS_PER_TILE, ROWS_PER_TILE)],
        out_hbm.at[cid, pl.ds(sid * ROWS_PER_TILE, ROWS_PER_TILE)],
    )


def _dinv_body(degp_ref, o_ref):
    o_ref[...] = lax.rsqrt(degp_ref[0] + degp_ref[1] + 1.0)


def _mm_scale_body(x_ref, w_ref, dv_ref, o_ref):
    o_ref[...] = jnp.dot(x_ref[...], w_ref[...],
                         preferred_element_type=jnp.float32) * dv_ref[...]


def _mid_body(a0_ref, a1_ref, hp_ref, dv_ref, b_ref, w_ref, o_ref):
    agg = a0_ref[...] + a1_ref[...]
    t = dv_ref[...] * (agg + hp_ref[...]) + b_ref[...]
    h = jnp.maximum(t, 0.0)
    o_ref[...] = jnp.dot(h, w_ref[...],
                         preferred_element_type=jnp.float32) * dv_ref[...]


def _fin_body(a0_ref, a1_ref, hp_ref, dv_ref, b_ref, o_ref):
    agg = a0_ref[...] + a1_ref[...]
    o_ref[...] = dv_ref[...] * (agg + hp_ref[...]) + b_ref[...]


_RB = 1000


def kernel(x, edge_index, W1, b1, W2, b2):
    src = edge_index[0].astype(jnp.int32)
    dst = edge_index[1].astype(jnp.int32)
    pad = jnp.full((E_PAD - N_EDGES,), N_NODES, jnp.int32)
    src_pad = jnp.concatenate([src, pad])
    dst_pad = jnp.concatenate([dst, pad])
    dstg32 = dst_pad.reshape(N_TILES, DEG_CHUNKS, CHUNK)
    srcg = src_pad.reshape(N_TILES, 2, HALF_CHUNKS, CHUNK)
    dstg = dst_pad.reshape(N_TILES, 2, HALF_CHUNKS, CHUNK)
    pad_rows = jnp.zeros((ACC_ROWS - N_NODES, D), jnp.float32)

    degp = _deg_kernel(dstg32)[:, :, 0]
    dinv2d = pl.pallas_call(
        _dinv_body,
        out_shape=jax.ShapeDtypeStruct((ACC_ROWS // D, D), jnp.float32),
    )(degp.reshape(2, ACC_ROWS // D, D))
    dinv_col = dinv2d.reshape(ACC_ROWS)[:N_NODES][:, None]

    h1p = pl.pallas_call(
        _mm_scale_body,
        grid=(N_NODES // _RB,),
        in_specs=[
            pl.BlockSpec((_RB, D), lambda i: (i, 0)),
            pl.BlockSpec((D, D), lambda i: (0, 0)),
            pl.BlockSpec((_RB, 1), lambda i: (i, 0)),
        ],
        out_specs=pl.BlockSpec((_RB, D), lambda i: (i, 0)),
        out_shape=jax.ShapeDtypeStruct((N_NODES, D), jnp.float32),
    )(x, W1, dinv_col)
    h1ext = jnp.concatenate([h1p, pad_rows])
    acc1 = _agg_kernel(h1ext, srcg, dstg)

    h2p = pl.pallas_call(
        _mid_body,
        grid=(N_NODES // _RB,),
        in_specs=[
            pl.BlockSpec((_RB, D), lambda i: (i, 0)),
            pl.BlockSpec((_RB, D), lambda i: (i, 0)),
            pl.BlockSpec((_RB, D), lambda i: (i, 0)),
            pl.BlockSpec((_RB, 1), lambda i: (i, 0)),
            pl.BlockSpec((1, D), lambda i: (0, 0)),
            pl.BlockSpec((D, D), lambda i: (0, 0)),
        ],
        out_specs=pl.BlockSpec((_RB, D), lambda i: (i, 0)),
        out_shape=jax.ShapeDtypeStruct((N_NODES, D), jnp.float32),
    )(acc1[0], acc1[1], h1p, dinv_col, b1.reshape(1, D), W2)
    h2ext = jnp.concatenate([h2p, pad_rows])
    acc2 = _agg_kernel(h2ext, srcg, dstg)

    out = pl.pallas_call(
        _fin_body,
        grid=(N_NODES // _RB,),
        in_specs=[
            pl.BlockSpec((_RB, D), lambda i: (i, 0)),
            pl.BlockSpec((_RB, D), lambda i: (i, 0)),
            pl.BlockSpec((_RB, D), lambda i: (i, 0)),
            pl.BlockSpec((_RB, 1), lambda i: (i, 0)),
            pl.BlockSpec((1, D), lambda i: (0, 0)),
        ],
        out_specs=pl.BlockSpec((_RB, D), lambda i: (i, 0)),
        out_shape=jax.ShapeDtypeStruct((N_NODES, D), jnp.float32),
    )(acc2[0], acc2[1], h2p, dinv_col, b2.reshape(1, D))
    return out
